# Initial kernel scaffold; baseline (speedup 1.0000x reference)
import jax
import jax.numpy as jnp
from jax.experimental import pallas as pl


def kernel(x, edge_index, edge_attr, batch, y, node_W, node_b, eenc_W, eenc_b, pna_ee_W, pna_ee_b, pre_W, pre_b, post_W, post_b, lin_W, lin_b, bn_g, bn_b, lin1_W, lin1_b, lin3_W, lin3_b):
    raise NotImplementedError("write your pallas kernel here")


# speedup vs baseline: 30.5027x; 30.5027x over previous
import dataclasses
import functools

import jax
import jax.numpy as jnp
import numpy as np
from jax import lax
from jax.experimental import pallas as pl
from jax.experimental.pallas import tpu as pltpu
from jax.experimental.pallas import tpu_sc as plsc

N = 10000
E = 160000
H = 64
L = 2
T = 4
G = 128
AVG_LOG = float(np.mean(np.log(np.arange(10, dtype=np.float64) + 1.0)))

NW = 32
NPW = 313
NPW_PAD = 320
WIN = 8000
NWIN = E // WIN
WPAD = WIN + 128
CH = 128

_HI = 3.0e38
_PREC = lax.Precision.HIGHEST


def _mesh():
  return plsc.VectorSubcoreMesh(core_axis_name="c", subcore_axis_name="s")


def _sc_params():
  cp = pltpu.CompilerParams()
  if "needs_layout_passes" in pltpu.CompilerParams.__dataclass_fields__:
    cp = dataclasses.replace(cp, needs_layout_passes=False)
  return cp


def _wid():
  return lax.axis_index("s") * 2 + lax.axis_index("c")


def _scalar_max(x):
  return x if x.ndim == 0 else jnp.max(x)


def _bucketize(src, dst, attr):
  @functools.partial(
      pl.kernel,
      out_type=[
          jax.ShapeDtypeStruct((NW * NWIN * WPAD,), jnp.int32),
          jax.ShapeDtypeStruct((NW * NWIN * 16,), jnp.int32),
      ],
      mesh=_mesh(),
      scratch_types=[
          pltpu.VMEM((WIN,), jnp.int32),
          pltpu.VMEM((WIN,), jnp.int32),
          pltpu.VMEM((WIN,), jnp.int32),
          pltpu.VMEM((WIN,), jnp.int32),
          pltpu.VMEM((NWIN * 16,), jnp.int32),
      ],
      compiler_params=_sc_params(),
  )
  def k(src_hbm, dst_hbm, attr_hbm, bk_hbm, cnt_hbm, sb, db, ab, ob, cb):
    wid = _wid()
    lo = wid * NPW
    hi = lo + NPW

    @pl.loop(0, NWIN)
    def _win(j):
      pltpu.sync_copy(src_hbm.at[pl.ds(j * WIN, WIN)], sb)
      pltpu.sync_copy(dst_hbm.at[pl.ds(j * WIN, WIN)], db)
      pltpu.sync_copy(attr_hbm.at[pl.ds(j * WIN, WIN)], ab)

      def body(g, off):
        b = pl.multiple_of(g * 16, 16)
        d = db[pl.ds(b, 16)]
        s = sb[pl.ds(b, 16)]
        a = ab[pl.ds(b, 16)]
        m = (d >= lo) & (d < hi)
        packed = s | ((d - lo) << 14) | (a << 23)
        plsc.store_compressed(ob.at[pl.ds(off, 16)], packed, mask=m)
        return off + _scalar_max(plsc.all_reduce_population_count(m))

      off = lax.fori_loop(0, WIN // 16, body, jnp.int32(0))
      cb[pl.ds(pl.multiple_of(j * 16, 16), 16)] = jnp.full((16,), off,
                                                          jnp.int32)
      pltpu.sync_copy(ob, bk_hbm.at[pl.ds((wid * NWIN + j) * WPAD, WIN)])

    pltpu.sync_copy(cb, cnt_hbm.at[pl.ds(wid * (NWIN * 16), NWIN * 16)])

  return k(src, dst, attr)


def _tower_pass(p_flat, ctab, buckets, counts):
  AW = NPW_PAD * H
  acc_t = jax.ShapeDtypeStruct((T * NW * AW,), jnp.float32)

  @functools.partial(
      pl.kernel,
      out_type=[acc_t, acc_t, acc_t, acc_t,
                jax.ShapeDtypeStruct((NW * NPW_PAD,), jnp.int32)],
      mesh=_mesh(),
      scratch_types=[
          pltpu.VMEM((AW,), jnp.float32),
          pltpu.VMEM((AW,), jnp.float32),
          pltpu.VMEM((AW,), jnp.float32),
          pltpu.VMEM((AW,), jnp.float32),
          pltpu.VMEM((NPW_PAD,), jnp.int32),
          pltpu.VMEM((NWIN * 16,), jnp.int32),
          pltpu.VMEM((CH,), jnp.int32),
          pltpu.VMEM((CH,), jnp.int32),
          pltpu.VMEM((CH, 128), jnp.float32),
          pltpu.VMEM((4 * H,), jnp.float32),
      ],
      compiler_params=_sc_params(),
  )
  def k(p_hbm, c_hbm, bk_hbm, cnt_hbm,
        s_out, q_out, mn_out, mx_out, deg_out,
        accs, accq, accmn, accmx, cntv, cw, pk, idxb, rows, ct):
    wid = _wid()
    pltpu.sync_copy(cnt_hbm.at[pl.ds(wid * (NWIN * 16), NWIN * 16)], cw)

    zi = jnp.zeros((16,), jnp.int32)

    @pl.loop(0, NPW_PAD // 16)
    def _zc(i):
      cntv[pl.ds(pl.multiple_of(i * 16, 16), 16)] = zi

    for t in range(T):
      pltpu.sync_copy(c_hbm.at[pl.ds(t * 4 * H, 4 * H)], ct)

      zf = jnp.zeros((16,), jnp.float32)
      pinf = jnp.full((16,), _HI, jnp.float32)
      ninf = jnp.full((16,), -_HI, jnp.float32)

      @pl.loop(0, AW // 16)
      def _init(r):
        sl = pl.ds(pl.multiple_of(r * 16, 16), 16)
        accs[sl] = zf
        accq[sl] = zf
        accmn[sl] = pinf
        accmx[sl] = ninf

      @pl.loop(0, NWIN)
      def _win(j):
        cnt_j = cw[pl.ds(pl.multiple_of(j * 16, 16), 16)][0]
        nch = (cnt_j + CH - 1) // CH

        def chunk(kk, _):
          base = kk * CH
          pltpu.sync_copy(
              bk_hbm.at[pl.ds((wid * NWIN + j) * WPAD + base, CH)], pk)

          @pl.loop(0, CH // 16)
          def _ib(v):
            b = pl.multiple_of(v * 16, 16)
            w = pk[pl.ds(b, 16)]
            sidx = jnp.minimum(w & 0x3FFF, N - 1) + t * N
            idxb[pl.ds(b, 16)] = sidx
            if t == 0:
              valid = (base + b + lax.iota(jnp.int32, 16)) < cnt_j
              dl16 = (w >> 14) & 511
              plsc.addupdate_scatter(cntv, [dl16],
                                     jnp.ones((16,), jnp.int32), mask=valid)

          pltpu.sync_copy(p_hbm.at[idxb], rows)
          ec = jnp.minimum(CH, cnt_j - base)

          def egrp(g, _2):
            b16 = pl.multiple_of(g * 16, 16)
            wvec = pk[pl.ds(b16, 16)]
            for u in range(16):
              wv = wvec[u]
              i = b16 + u

              @pl.when(i < ec)
              def _do(wv=wv, i=i):
                dlo = ((wv >> 14) & 511) * H
                ato = ((wv >> 23) & 3) * H
                for cg in range(H // 16):
                  co = cg * 16
                  m = rows[i, pl.ds(co, 16)] + ct[pl.ds(ato + co, 16)]
                  sl = pl.ds(dlo + co, 16)
                  accs[sl] = accs[sl] + m
                  accq[sl] = accq[sl] + m * m
                  accmn[sl] = jnp.minimum(accmn[sl], m)
                  accmx[sl] = jnp.maximum(accmx[sl], m)

            return 0

          lax.fori_loop(0, (ec + 15) // 16, egrp, 0)
          return 0

        lax.fori_loop(0, nch, chunk, 0)

      pltpu.sync_copy(accs, s_out.at[pl.ds((t * NW + wid) * AW, AW)])
      pltpu.sync_copy(accq, q_out.at[pl.ds((t * NW + wid) * AW, AW)])
      pltpu.sync_copy(accmn, mn_out.at[pl.ds((t * NW + wid) * AW, AW)])
      pltpu.sync_copy(accmx, mx_out.at[pl.ds((t * NW + wid) * AW, AW)])

    pltpu.sync_copy(cntv, deg_out.at[pl.ds(wid * NPW_PAD, NPW_PAD)])

  return k(p_flat, ctab, buckets, counts)


def _node_enc(x_p, w_p, b):
  def body(x_ref, w_ref, b_ref, o_ref):
    o_ref[...] = jnp.dot(x_ref[...], w_ref[...], precision=_PREC) + b_ref[...]

  return pl.pallas_call(
      body,
      out_shape=jax.ShapeDtypeStruct((N, H), jnp.float32),
  )(x_p, w_p, b)


def _ctables(eenc_W, eenc_b, ee_W, ee_b, we4):
  def body(ew_ref, eb_ref, w_ref, b_ref, we_ref, o_ref):
    ea_h = ew_ref[...] + eb_ref[...]
    for l in range(L):
      ea = jnp.dot(ea_h, w_ref[l], precision=_PREC) + b_ref[l][None]
      for t in range(T):
        o_ref[l, t] = jnp.dot(ea, we_ref[l, t], precision=_PREC)

  return pl.pallas_call(
      body,
      out_shape=jax.ShapeDtypeStruct((L, T, 4, H), jnp.float32),
  )(eenc_W, eenc_b.reshape(1, H), ee_W, ee_b, we4)


def _pre(h, wi3, wj3, pre_b):
  def body(h_ref, wi_ref, wj_ref, b_ref, a_ref, p_ref):
    hh = h_ref[...]
    a_ref[0] = jnp.dot(hh, wi_ref[0], precision=_PREC) + b_ref[0]
    pp = jnp.dot(hh, wj_ref[0], precision=_PREC)
    p_ref[0] = jnp.concatenate(
        [pp, jnp.zeros((N, 128 - H), jnp.float32)], axis=-1)

  return pl.pallas_call(
      body,
      grid=(T,),
      in_specs=[
          pl.BlockSpec((N, H), lambda t: (0, 0)),
          pl.BlockSpec((1, H, H), lambda t: (t, 0, 0)),
          pl.BlockSpec((1, H, H), lambda t: (t, 0, 0)),
          pl.BlockSpec((1, 1, H), lambda t: (t, 0, 0)),
      ],
      out_specs=[
          pl.BlockSpec((1, N, H), lambda t: (t, 0, 0)),
          pl.BlockSpec((1, N, 128), lambda t: (t, 0, 0)),
      ],
      out_shape=[
          jax.ShapeDtypeStruct((T, N, H), jnp.float32),
          jax.ShapeDtypeStruct((T, N, 128), jnp.float32),
      ],
  )(h, wi3, wj3, pre_b.reshape(T, 1, H))


def _post(h, a3, s3, q3, mn3, mx3, deg, wx, w1, w2, w3, pb):
  def body(h_ref, a_ref, s_ref, q_ref, mn_ref, mx_ref, d_ref,
           wx_ref, w1_ref, w2_ref, w3_ref, pb_ref, o_ref):
    d = d_ref[...]
    degc = jnp.maximum(d, 1.0)
    a = a_ref[0]
    s = s_ref[0]
    q = q_ref[0]
    s1 = d * a + s
    mean = s1 / degc
    msq = (d * a * a + 2.0 * a * s + q) / degc
    var = jnp.maximum(msq - mean * mean, 0.0)
    std = jnp.sqrt(var + 1e-5)
    has = d > 0.0
    mn = jnp.where(has, a + mn_ref[0], 0.0)
    mx = jnp.where(has, a + mx_ref[0], 0.0)
    dl = jnp.log(degc + 1.0)
    sc1 = dl / AVG_LOG
    sc2 = AVG_LOG / dl
    agg = jnp.concatenate([mean, mn, mx, std], axis=-1)
    out = (jnp.dot(h_ref[...], wx_ref[0], precision=_PREC)
           + jnp.dot(agg, w1_ref[0], precision=_PREC)
           + sc1 * jnp.dot(agg, w2_ref[0], precision=_PREC)
           + sc2 * jnp.dot(agg, w3_ref[0], precision=_PREC)
           + pb_ref[0])
    o_ref[0] = out

  nto = H // T
  nb = 2000
  return pl.pallas_call(
      body,
      grid=(T, N // nb),
      in_specs=[
          pl.BlockSpec((nb, H), lambda t, i: (i, 0)),
          pl.BlockSpec((1, nb, H), lambda t, i: (t, i, 0)),
          pl.BlockSpec((1, nb, H), lambda t, i: (t, i, 0)),
          pl.BlockSpec((1, nb, H), lambda t, i: (t, i, 0)),
          pl.BlockSpec((1, nb, H), lambda t, i: (t, i, 0)),
          pl.BlockSpec((1, nb, H), lambda t, i: (t, i, 0)),
          pl.BlockSpec((nb, 1), lambda t, i: (i, 0)),
          pl.BlockSpec((1, H, nto), lambda t, i: (t, 0, 0)),
          pl.BlockSpec((1, 4 * H, nto), lambda t, i: (t, 0, 0)),
          pl.BlockSpec((1, 4 * H, nto), lambda t, i: (t, 0, 0)),
          pl.BlockSpec((1, 4 * H, nto), lambda t, i: (t, 0, 0)),
          pl.BlockSpec((1, 1, nto), lambda t, i: (t, 0, 0)),
      ],
      out_specs=pl.BlockSpec((1, nb, nto), lambda t, i: (t, i, 0)),
      out_shape=jax.ShapeDtypeStruct((T, N, nto), jnp.float32),
  )(h, a3, s3, q3, mn3, mx3, deg, wx, w1, w2, w3, pb.reshape(T, 1, nto))


def _lin_bn_res(y4, h_in, lw, lb, g, b):
  def body(y_ref, h_ref, w_ref, b_ref, g_ref, bb_ref, o_ref):
    yy = jnp.concatenate([y_ref[t] for t in range(T)], axis=-1)
    y0 = jnp.dot(yy, w_ref[...], precision=_PREC) + b_ref[...]
    m = jnp.mean(y0, axis=0, keepdims=True)
    v = jnp.mean((y0 - m) * (y0 - m), axis=0, keepdims=True)
    hn = (y0 - m) / jnp.sqrt(v + 1e-5) * g_ref[...] + bb_ref[...]
    o_ref[...] = jnp.maximum(hn, 0.0) + h_ref[...]

  return pl.pallas_call(
      body,
      out_shape=jax.ShapeDtypeStruct((N, H), jnp.float32),
  )(y4, h_in, lw, lb.reshape(1, H), g.reshape(1, H), b.reshape(1, H))


def _pool_head(h, batch2, y2, w1, b1, w3, b3):
  def body(h_ref, bt_ref, y_ref, w1_ref, b1_ref, w3_ref, b3_ref, o_ref):
    gids = lax.broadcasted_iota(jnp.int32, (1, G), 1)
    oh = (bt_ref[...] == gids).astype(jnp.float32)
    pooled = lax.dot_general(oh, h_ref[...], (((0,), (0,)), ((), ())),
                             precision=_PREC)
    xc = jnp.maximum(jnp.dot(pooled, w1_ref[...], precision=_PREC)
                     + b1_ref[...], 0.0)
    pred = jnp.dot(xc, w3_ref[...], precision=_PREC) + b3_ref[...]
    dd = jnp.abs(pred - y_ref[...])
    beta = 0.5
    ls = jnp.where(dd < beta, 0.5 * dd * dd / beta, dd - 0.5 * beta)
    o_ref[...] = jnp.mean(ls).reshape(1, 1)

  return pl.pallas_call(
      body,
      out_shape=jax.ShapeDtypeStruct((1, 1), jnp.float32),
  )(h, batch2, y2, w1, b1.reshape(1, H), w3, b3.reshape(1, 1))


def kernel(x, edge_index, edge_attr, batch, y, node_W, node_b, eenc_W,
           eenc_b, pna_ee_W, pna_ee_b, pre_W, pre_b, post_W, post_b, lin_W,
           lin_b, bn_g, bn_b, lin1_W, lin1_b, lin3_W, lin3_b):
  f32 = jnp.float32
  src = edge_index[0].astype(jnp.int32)
  dst = edge_index[1].astype(jnp.int32)
  attr = edge_attr.astype(jnp.int32)

  buckets, counts = _bucketize(src, dst, attr)

  x_p = jnp.pad(x.astype(f32), ((0, 0), (0, 7)))
  w_p = jnp.pad(node_W.astype(f32), ((0, 7), (0, 0)))
  h = _node_enc(x_p, w_p, node_b.astype(f32).reshape(1, H))

  we4 = pre_W[:, :, 2 * H:3 * H, :].astype(f32)
  c_all = _ctables(eenc_W.astype(f32), eenc_b.astype(f32),
                   pna_ee_W.astype(f32), pna_ee_b.astype(f32), we4)

  for l in range(L):
    wi3 = pre_W[l, :, :H, :].astype(f32)
    wj3 = pre_W[l, :, H:2 * H, :].astype(f32)
    a3, p3 = _pre(h, wi3, wj3, pre_b[l].astype(f32))

    s4, q4, mn4, mx4, cnt = _tower_pass(
        p3.reshape(T * N, 128), c_all[l].reshape(-1), buckets, counts)

    def _trim(z):
      z = z.reshape(T, NW, NPW_PAD, H)
      return z[:, :, :NPW].reshape(T, NW * NPW, H)[:, :N]

    s3 = _trim(s4)
    q3 = _trim(q4)
    mn3 = _trim(mn4)
    mx3 = _trim(mx4)
    deg = cnt.reshape(NW, NPW_PAD)[:, :NPW].reshape(-1)[:N]
    deg = deg.astype(f32).reshape(N, 1)

    wx = post_W[l, :, :H, :].astype(f32)
    w1 = post_W[l, :, H:5 * H, :].astype(f32)
    w2 = post_W[l, :, 5 * H:9 * H, :].astype(f32)
    w3 = post_W[l, :, 9 * H:13 * H, :].astype(f32)

    out64 = _post(h, a3, s3, q3, mn3, mx3, deg, wx, w1, w2, w3,
                  post_b[l].astype(f32))
    h = _lin_bn_res(out64, h, lin_W[l].astype(f32), lin_b[l].astype(f32),
                    bn_g[l].astype(f32), bn_b[l].astype(f32))

  loss = _pool_head(h, batch.astype(jnp.int32).reshape(N, 1),
                    y.astype(f32).reshape(G, 1), lin1_W.astype(f32),
                    lin1_b.astype(f32), lin3_W.astype(f32),
                    lin3_b.astype(f32))
  loss = loss.reshape(())
  return (loss, loss)


# speedup vs baseline: 31.6808x; 1.0386x over previous
import dataclasses
import functools

import jax
import jax.numpy as jnp
import numpy as np
from jax import lax
from jax.experimental import pallas as pl
from jax.experimental.pallas import tpu as pltpu
from jax.experimental.pallas import tpu_sc as plsc

N = 10000
E = 160000
H = 64
L = 2
T = 4
G = 128
AVG_LOG = float(np.mean(np.log(np.arange(10, dtype=np.float64) + 1.0)))

NW = 32
NPW = 313
NPW_PAD = 320
WIN = 8000
NWIN = E // WIN
WPAD = WIN + 128
CH = 128

_HI = 3.0e38
_PREC = lax.Precision.HIGHEST


def _mesh():
  return plsc.VectorSubcoreMesh(core_axis_name="c", subcore_axis_name="s")


def _sc_params():
  cp = pltpu.CompilerParams()
  if "needs_layout_passes" in pltpu.CompilerParams.__dataclass_fields__:
    cp = dataclasses.replace(cp, needs_layout_passes=False)
  return cp


def _wid():
  return lax.axis_index("s") * 2 + lax.axis_index("c")


def _scalar_max(x):
  return x if x.ndim == 0 else jnp.max(x)


def _bucketize(src, dst, attr):
  @functools.partial(
      pl.kernel,
      out_type=[
          jax.ShapeDtypeStruct((NW * NWIN * WPAD,), jnp.int32),
          jax.ShapeDtypeStruct((NW * NWIN * 16,), jnp.int32),
      ],
      mesh=_mesh(),
      scratch_types=[
          pltpu.VMEM((WIN,), jnp.int32),
          pltpu.VMEM((WIN,), jnp.int32),
          pltpu.VMEM((WIN,), jnp.int32),
          pltpu.VMEM((WIN,), jnp.int32),
          pltpu.VMEM((NWIN * 16,), jnp.int32),
      ],
      compiler_params=_sc_params(),
  )
  def k(src_hbm, dst_hbm, attr_hbm, bk_hbm, cnt_hbm, sb, db, ab, ob, cb):
    wid = _wid()
    lo = wid * NPW
    hi = lo + NPW

    @pl.loop(0, NWIN)
    def _win(j):
      pltpu.sync_copy(src_hbm.at[pl.ds(j * WIN, WIN)], sb)
      pltpu.sync_copy(dst_hbm.at[pl.ds(j * WIN, WIN)], db)
      pltpu.sync_copy(attr_hbm.at[pl.ds(j * WIN, WIN)], ab)

      def body(g, off):
        b = pl.multiple_of(g * 16, 16)
        d = db[pl.ds(b, 16)]
        s = sb[pl.ds(b, 16)]
        a = ab[pl.ds(b, 16)]
        m = (d >= lo) & (d < hi)
        packed = s | ((d - lo) << 14) | (a << 23)
        plsc.store_compressed(ob.at[pl.ds(off, 16)], packed, mask=m)
        return off + _scalar_max(plsc.all_reduce_population_count(m))

      off = lax.fori_loop(0, WIN // 16, body, jnp.int32(0))
      cb[pl.ds(pl.multiple_of(j * 16, 16), 16)] = jnp.full((16,), off,
                                                          jnp.int32)
      pltpu.sync_copy(ob, bk_hbm.at[pl.ds((wid * NWIN + j) * WPAD, WIN)])

    pltpu.sync_copy(cb, cnt_hbm.at[pl.ds(wid * (NWIN * 16), NWIN * 16)])

  return k(src, dst, attr)


def _tower_pass(p_flat, ctab, buckets, counts):
  AW = NPW_PAD * H
  acc_t = jax.ShapeDtypeStruct((T * NW * AW,), jnp.float32)

  @functools.partial(
      pl.kernel,
      out_type=[acc_t, acc_t, acc_t, acc_t,
                jax.ShapeDtypeStruct((NW * NPW_PAD,), jnp.int32)],
      mesh=_mesh(),
      scratch_types=[
          pltpu.VMEM((AW,), jnp.float32),
          pltpu.VMEM((AW,), jnp.float32),
          pltpu.VMEM((AW,), jnp.float32),
          pltpu.VMEM((AW,), jnp.float32),
          pltpu.VMEM((NPW_PAD,), jnp.int32),
          pltpu.VMEM((NWIN * 16,), jnp.int32),
          pltpu.VMEM((CH,), jnp.int32),
          pltpu.VMEM((CH,), jnp.int32),
          pltpu.VMEM((CH, 128), jnp.float32),
          pltpu.VMEM((4 * H,), jnp.float32),
      ],
      compiler_params=_sc_params(),
  )
  def k(p_hbm, c_hbm, bk_hbm, cnt_hbm,
        s_out, q_out, mn_out, mx_out, deg_out,
        accs, accq, accmn, accmx, cntv, cw, pk, idxb, rows, ct):
    wid = _wid()
    pltpu.sync_copy(cnt_hbm.at[pl.ds(wid * (NWIN * 16), NWIN * 16)], cw)

    zi = jnp.zeros((16,), jnp.int32)

    @pl.loop(0, NPW_PAD // 16)
    def _zc(i):
      cntv[pl.ds(pl.multiple_of(i * 16, 16), 16)] = zi

    for t in range(T):
      pltpu.sync_copy(c_hbm.at[pl.ds(t * 4 * H, 4 * H)], ct)

      zf = jnp.zeros((16,), jnp.float32)
      pinf = jnp.full((16,), _HI, jnp.float32)
      ninf = jnp.full((16,), -_HI, jnp.float32)

      @pl.loop(0, AW // 16)
      def _init(r):
        sl = pl.ds(pl.multiple_of(r * 16, 16), 16)
        accs[sl] = zf
        accq[sl] = zf
        accmn[sl] = pinf
        accmx[sl] = ninf

      @pl.loop(0, NWIN)
      def _win(j):
        cnt_j = cw[pl.ds(pl.multiple_of(j * 16, 16), 16)][0]
        nch = (cnt_j + CH - 1) // CH

        def chunk(kk, _):
          base = kk * CH
          pltpu.sync_copy(
              bk_hbm.at[pl.ds((wid * NWIN + j) * WPAD + base, CH)], pk)

          @pl.loop(0, CH // 16)
          def _ib(v):
            b = pl.multiple_of(v * 16, 16)
            w = pk[pl.ds(b, 16)]
            sidx = jnp.minimum(w & 0x3FFF, N - 1) + t * N
            idxb[pl.ds(b, 16)] = sidx
            if t == 0:
              valid = (base + b + lax.iota(jnp.int32, 16)) < cnt_j
              dl16 = (w >> 14) & 511
              plsc.addupdate_scatter(cntv, [dl16],
                                     jnp.ones((16,), jnp.int32), mask=valid)

          pltpu.sync_copy(p_hbm.at[idxb], rows)
          ec = jnp.minimum(CH, cnt_j - base)

          def egrp(g, _2):
            b16 = pl.multiple_of(g * 16, 16)
            wvec = pk[pl.ds(b16, 16)]
            for u in range(16):
              wv = wvec[u]
              i = b16 + u
              dl = jnp.where(i < ec, jnp.minimum((wv >> 14) & 511, NPW), NPW)
              dlo = dl * H
              ato = ((wv >> 23) & 3) * H
              for cg in range(H // 16):
                co = cg * 16
                m = rows[i, pl.ds(co, 16)] + ct[pl.ds(ato + co, 16)]
                sl = pl.ds(dlo + co, 16)
                accs[sl] = accs[sl] + m
                accq[sl] = accq[sl] + m * m
                accmn[sl] = jnp.minimum(accmn[sl], m)
                accmx[sl] = jnp.maximum(accmx[sl], m)

            return 0

          lax.fori_loop(0, (ec + 15) // 16, egrp, 0)
          return 0

        lax.fori_loop(0, nch, chunk, 0)

      pltpu.sync_copy(accs, s_out.at[pl.ds((t * NW + wid) * AW, AW)])
      pltpu.sync_copy(accq, q_out.at[pl.ds((t * NW + wid) * AW, AW)])
      pltpu.sync_copy(accmn, mn_out.at[pl.ds((t * NW + wid) * AW, AW)])
      pltpu.sync_copy(accmx, mx_out.at[pl.ds((t * NW + wid) * AW, AW)])

    pltpu.sync_copy(cntv, deg_out.at[pl.ds(wid * NPW_PAD, NPW_PAD)])

  return k(p_flat, ctab, buckets, counts)


def _node_enc(x_p, w_p, b):
  def body(x_ref, w_ref, b_ref, o_ref):
    o_ref[...] = jnp.dot(x_ref[...], w_ref[...], precision=_PREC) + b_ref[...]

  return pl.pallas_call(
      body,
      out_shape=jax.ShapeDtypeStruct((N, H), jnp.float32),
  )(x_p, w_p, b)


def _ctables(eenc_W, eenc_b, ee_W, ee_b, we4):
  def body(ew_ref, eb_ref, w_ref, b_ref, we_ref, o_ref):
    ea_h = ew_ref[...] + eb_ref[...]
    for l in range(L):
      ea = jnp.dot(ea_h, w_ref[l], precision=_PREC) + b_ref[l][None]
      for t in range(T):
        o_ref[l, t] = jnp.dot(ea, we_ref[l, t], precision=_PREC)

  return pl.pallas_call(
      body,
      out_shape=jax.ShapeDtypeStruct((L, T, 4, H), jnp.float32),
  )(eenc_W, eenc_b.reshape(1, H), ee_W, ee_b, we4)


def _pre(h, wi3, wj3, pre_b):
  def body(h_ref, wi_ref, wj_ref, b_ref, a_ref, p_ref):
    hh = h_ref[...]
    a_ref[0] = jnp.dot(hh, wi_ref[0], precision=_PREC) + b_ref[0]
    pp = jnp.dot(hh, wj_ref[0], precision=_PREC)
    p_ref[0] = jnp.concatenate(
        [pp, jnp.zeros((N, 128 - H), jnp.float32)], axis=-1)

  return pl.pallas_call(
      body,
      grid=(T,),
      in_specs=[
          pl.BlockSpec((N, H), lambda t: (0, 0)),
          pl.BlockSpec((1, H, H), lambda t: (t, 0, 0)),
          pl.BlockSpec((1, H, H), lambda t: (t, 0, 0)),
          pl.BlockSpec((1, 1, H), lambda t: (t, 0, 0)),
      ],
      out_specs=[
          pl.BlockSpec((1, N, H), lambda t: (t, 0, 0)),
          pl.BlockSpec((1, N, 128), lambda t: (t, 0, 0)),
      ],
      out_shape=[
          jax.ShapeDtypeStruct((T, N, H), jnp.float32),
          jax.ShapeDtypeStruct((T, N, 128), jnp.float32),
      ],
  )(h, wi3, wj3, pre_b.reshape(T, 1, H))


def _post(h, a3, s3, q3, mn3, mx3, deg, wx, w1, w2, w3, pb):
  def body(h_ref, a_ref, s_ref, q_ref, mn_ref, mx_ref, d_ref,
           wx_ref, w1_ref, w2_ref, w3_ref, pb_ref, o_ref):
    d = d_ref[...]
    degc = jnp.maximum(d, 1.0)
    a = a_ref[0]
    s = s_ref[0]
    q = q_ref[0]
    s1 = d * a + s
    mean = s1 / degc
    msq = (d * a * a + 2.0 * a * s + q) / degc
    var = jnp.maximum(msq - mean * mean, 0.0)
    std = jnp.sqrt(var + 1e-5)
    has = d > 0.0
    mn = jnp.where(has, a + mn_ref[0], 0.0)
    mx = jnp.where(has, a + mx_ref[0], 0.0)
    dl = jnp.log(degc + 1.0)
    sc1 = dl / AVG_LOG
    sc2 = AVG_LOG / dl
    agg = jnp.concatenate([mean, mn, mx, std], axis=-1)
    out = (jnp.dot(h_ref[...], wx_ref[0], precision=_PREC)
           + jnp.dot(agg, w1_ref[0], precision=_PREC)
           + sc1 * jnp.dot(agg, w2_ref[0], precision=_PREC)
           + sc2 * jnp.dot(agg, w3_ref[0], precision=_PREC)
           + pb_ref[0])
    o_ref[0] = out

  nto = H // T
  nb = 2000
  return pl.pallas_call(
      body,
      grid=(T, N // nb),
      in_specs=[
          pl.BlockSpec((nb, H), lambda t, i: (i, 0)),
          pl.BlockSpec((1, nb, H), lambda t, i: (t, i, 0)),
          pl.BlockSpec((1, nb, H), lambda t, i: (t, i, 0)),
          pl.BlockSpec((1, nb, H), lambda t, i: (t, i, 0)),
          pl.BlockSpec((1, nb, H), lambda t, i: (t, i, 0)),
          pl.BlockSpec((1, nb, H), lambda t, i: (t, i, 0)),
          pl.BlockSpec((nb, 1), lambda t, i: (i, 0)),
          pl.BlockSpec((1, H, nto), lambda t, i: (t, 0, 0)),
          pl.BlockSpec((1, 4 * H, nto), lambda t, i: (t, 0, 0)),
          pl.BlockSpec((1, 4 * H, nto), lambda t, i: (t, 0, 0)),
          pl.BlockSpec((1, 4 * H, nto), lambda t, i: (t, 0, 0)),
          pl.BlockSpec((1, 1, nto), lambda t, i: (t, 0, 0)),
      ],
      out_specs=pl.BlockSpec((1, nb, nto), lambda t, i: (t, i, 0)),
      out_shape=jax.ShapeDtypeStruct((T, N, nto), jnp.float32),
  )(h, a3, s3, q3, mn3, mx3, deg, wx, w1, w2, w3, pb.reshape(T, 1, nto))


def _lin_bn_res(y4, h_in, lw, lb, g, b):
  def body(y_ref, h_ref, w_ref, b_ref, g_ref, bb_ref, o_ref):
    yy = jnp.concatenate([y_ref[t] for t in range(T)], axis=-1)
    y0 = jnp.dot(yy, w_ref[...], precision=_PREC) + b_ref[...]
    m = jnp.mean(y0, axis=0, keepdims=True)
    v = jnp.mean((y0 - m) * (y0 - m), axis=0, keepdims=True)
    hn = (y0 - m) / jnp.sqrt(v + 1e-5) * g_ref[...] + bb_ref[...]
    o_ref[...] = jnp.maximum(hn, 0.0) + h_ref[...]

  return pl.pallas_call(
      body,
      out_shape=jax.ShapeDtypeStruct((N, H), jnp.float32),
  )(y4, h_in, lw, lb.reshape(1, H), g.reshape(1, H), b.reshape(1, H))


def _pool_head(h, batch2, y2, w1, b1, w3, b3):
  def body(h_ref, bt_ref, y_ref, w1_ref, b1_ref, w3_ref, b3_ref, o_ref):
    gids = lax.broadcasted_iota(jnp.int32, (1, G), 1)
    oh = (bt_ref[...] == gids).astype(jnp.float32)
    pooled = lax.dot_general(oh, h_ref[...], (((0,), (0,)), ((), ())),
                             precision=_PREC)
    xc = jnp.maximum(jnp.dot(pooled, w1_ref[...], precision=_PREC)
                     + b1_ref[...], 0.0)
    pred = jnp.dot(xc, w3_ref[...], precision=_PREC) + b3_ref[...]
    dd = jnp.abs(pred - y_ref[...])
    beta = 0.5
    ls = jnp.where(dd < beta, 0.5 * dd * dd / beta, dd - 0.5 * beta)
    o_ref[...] = jnp.mean(ls).reshape(1, 1)

  return pl.pallas_call(
      body,
      out_shape=jax.ShapeDtypeStruct((1, 1), jnp.float32),
  )(h, batch2, y2, w1, b1.reshape(1, H), w3, b3.reshape(1, 1))


def kernel(x, edge_index, edge_attr, batch, y, node_W, node_b, eenc_W,
           eenc_b, pna_ee_W, pna_ee_b, pre_W, pre_b, post_W, post_b, lin_W,
           lin_b, bn_g, bn_b, lin1_W, lin1_b, lin3_W, lin3_b):
  f32 = jnp.float32
  src = edge_index[0].astype(jnp.int32)
  dst = edge_index[1].astype(jnp.int32)
  attr = edge_attr.astype(jnp.int32)

  buckets, counts = _bucketize(src, dst, attr)

  x_p = jnp.pad(x.astype(f32), ((0, 0), (0, 7)))
  w_p = jnp.pad(node_W.astype(f32), ((0, 7), (0, 0)))
  h = _node_enc(x_p, w_p, node_b.astype(f32).reshape(1, H))

  we4 = pre_W[:, :, 2 * H:3 * H, :].astype(f32)
  c_all = _ctables(eenc_W.astype(f32), eenc_b.astype(f32),
                   pna_ee_W.astype(f32), pna_ee_b.astype(f32), we4)

  for l in range(L):
    wi3 = pre_W[l, :, :H, :].astype(f32)
    wj3 = pre_W[l, :, H:2 * H, :].astype(f32)
    a3, p3 = _pre(h, wi3, wj3, pre_b[l].astype(f32))

    s4, q4, mn4, mx4, cnt = _tower_pass(
        p3.reshape(T * N, 128), c_all[l].reshape(-1), buckets, counts)

    def _trim(z):
      z = z.reshape(T, NW, NPW_PAD, H)
      return z[:, :, :NPW].reshape(T, NW * NPW, H)[:, :N]

    s3 = _trim(s4)
    q3 = _trim(q4)
    mn3 = _trim(mn4)
    mx3 = _trim(mx4)
    deg = cnt.reshape(NW, NPW_PAD)[:, :NPW].reshape(-1)[:N]
    deg = deg.astype(f32).reshape(N, 1)

    wx = post_W[l, :, :H, :].astype(f32)
    w1 = post_W[l, :, H:5 * H, :].astype(f32)
    w2 = post_W[l, :, 5 * H:9 * H, :].astype(f32)
    w3 = post_W[l, :, 9 * H:13 * H, :].astype(f32)

    out64 = _post(h, a3, s3, q3, mn3, mx3, deg, wx, w1, w2, w3,
                  post_b[l].astype(f32))
    h = _lin_bn_res(out64, h, lin_W[l].astype(f32), lin_b[l].astype(f32),
                    bn_g[l].astype(f32), bn_b[l].astype(f32))

  loss = _pool_head(h, batch.astype(jnp.int32).reshape(N, 1),
                    y.astype(f32).reshape(G, 1), lin1_W.astype(f32),
                    lin1_b.astype(f32), lin3_W.astype(f32),
                    lin3_b.astype(f32))
  loss = loss.reshape(())
  return (loss, loss)


# speedup vs baseline: 49.7454x; 1.5702x over previous
import dataclasses
import functools

import jax
import jax.numpy as jnp
import numpy as np
from jax import lax
from jax.experimental import pallas as pl
from jax.experimental.pallas import tpu as pltpu
from jax.experimental.pallas import tpu_sc as plsc

N = 10000
E = 160000
H = 64
L = 2
T = 4
G = 128
AVG_LOG = float(np.mean(np.log(np.arange(10, dtype=np.float64) + 1.0)))

NW = 32
NPW = 313
NPW_PAD = 320
WIN = 20000
NWIN = E // WIN
WPAD = WIN + 128
CH = 128

_HI = 3.0e38
_PREC = None


def _mesh():
  return plsc.VectorSubcoreMesh(core_axis_name="c", subcore_axis_name="s")


def _sc_params():
  cp = pltpu.CompilerParams()
  if "needs_layout_passes" in pltpu.CompilerParams.__dataclass_fields__:
    cp = dataclasses.replace(cp, needs_layout_passes=False)
  return cp


def _wid():
  return lax.axis_index("s") * 2 + lax.axis_index("c")


def _scalar_max(x):
  return x if x.ndim == 0 else jnp.max(x)


def _bucketize(src, dst, attr):
  @functools.partial(
      pl.kernel,
      out_type=[
          jax.ShapeDtypeStruct((NW * NWIN * WPAD,), jnp.int32),
          jax.ShapeDtypeStruct((NW * NWIN * 16,), jnp.int32),
      ],
      mesh=_mesh(),
      scratch_types=[
          pltpu.VMEM((WIN,), jnp.int32),
          pltpu.VMEM((WIN,), jnp.int32),
          pltpu.VMEM((WIN,), jnp.int32),
          pltpu.VMEM((WIN,), jnp.int32),
          pltpu.VMEM((NWIN * 16,), jnp.int32),
      ],
      compiler_params=_sc_params(),
  )
  def k(src_hbm, dst_hbm, attr_hbm, bk_hbm, cnt_hbm, sb, db, ab, ob, cb):
    wid = _wid()
    lo = wid * NPW
    hi = lo + NPW

    @pl.loop(0, NWIN)
    def _win(j):
      pltpu.sync_copy(src_hbm.at[pl.ds(j * WIN, WIN)], sb)
      pltpu.sync_copy(dst_hbm.at[pl.ds(j * WIN, WIN)], db)
      pltpu.sync_copy(attr_hbm.at[pl.ds(j * WIN, WIN)], ab)

      def body(g, off):
        b = pl.multiple_of(g * 16, 16)
        d = db[pl.ds(b, 16)]
        s = sb[pl.ds(b, 16)]
        a = ab[pl.ds(b, 16)]
        m = (d >= lo) & (d < hi)
        packed = s | ((d - lo) << 14) | (a << 23)
        plsc.store_compressed(ob.at[pl.ds(off, 16)], packed, mask=m)
        return off + _scalar_max(plsc.all_reduce_population_count(m))

      off = lax.fori_loop(0, WIN // 16, body, jnp.int32(0))
      cb[pl.ds(pl.multiple_of(j * 16, 16), 16)] = jnp.full((16,), off,
                                                          jnp.int32)
      pltpu.sync_copy(ob, bk_hbm.at[pl.ds((wid * NWIN + j) * WPAD, WIN)])

    pltpu.sync_copy(cb, cnt_hbm.at[pl.ds(wid * (NWIN * 16), NWIN * 16)])

  return k(src, dst, attr)


def _tower_pass(p_flat, ctab, buckets, counts):
  AW = NPW_PAD * H
  acc_t = jax.ShapeDtypeStruct((T * NW * AW,), jnp.float32)

  @functools.partial(
      pl.kernel,
      out_type=[acc_t, acc_t, acc_t, acc_t,
                jax.ShapeDtypeStruct((NW * NPW_PAD,), jnp.int32)],
      mesh=_mesh(),
      scratch_types=[
          pltpu.VMEM((AW,), jnp.float32),
          pltpu.VMEM((AW,), jnp.float32),
          pltpu.VMEM((AW,), jnp.float32),
          pltpu.VMEM((AW,), jnp.float32),
          pltpu.VMEM((NPW_PAD,), jnp.int32),
          pltpu.VMEM((NWIN * 16,), jnp.int32),
          pltpu.VMEM((2, CH), jnp.int32),
          pltpu.VMEM((2, CH), jnp.int32),
          pltpu.VMEM((2, CH, 128), jnp.float32),
          pltpu.VMEM((4 * H,), jnp.float32),
          pltpu.SemaphoreType.DMA,
          pltpu.SemaphoreType.DMA,
      ],
      compiler_params=_sc_params(),
  )
  def k(p_hbm, c_hbm, bk_hbm, cnt_hbm,
        s_out, q_out, mn_out, mx_out, deg_out,
        accs, accq, accmn, accmx, cntv, cw, pk2, idx2, rows2, ct,
        sem0, sem1):
    wid = _wid()
    sems = (sem0, sem1)
    pltpu.sync_copy(cnt_hbm.at[pl.ds(wid * (NWIN * 16), NWIN * 16)], cw)

    zi = jnp.zeros((16,), jnp.int32)

    @pl.loop(0, NPW_PAD // 16)
    def _zc(i):
      cntv[pl.ds(pl.multiple_of(i * 16, 16), 16)] = zi

    @pl.loop(0, T)
    def _tower(t):
      pltpu.sync_copy(c_hbm.at[pl.ds(t * 4 * H, 4 * H)], ct)

      zf = jnp.zeros((16,), jnp.float32)
      pinf = jnp.full((16,), _HI, jnp.float32)
      ninf = jnp.full((16,), -_HI, jnp.float32)

      @pl.loop(0, AW // 16)
      def _init(r):
        sl = pl.ds(pl.multiple_of(r * 16, 16), 16)
        accs[sl] = zf
        accq[sl] = zf
        accmn[sl] = pinf
        accmx[sl] = ninf

      @pl.loop(0, NWIN)
      def _win(j):
        cnt_j = cw[pl.ds(pl.multiple_of(j * 16, 16), 16)][0]
        nch = (cnt_j + CH - 1) // CH
        row0 = (wid * NWIN + j) * WPAD

        def fetch(kk, p):
          base = kk * CH
          pltpu.sync_copy(bk_hbm.at[pl.ds(row0 + base, CH)], pk2.at[p])

          @pl.loop(0, CH // 16)
          def _ib(v):
            b = pl.multiple_of(v * 16, 16)
            w = pk2[p, pl.ds(b, 16)]
            sidx = jnp.minimum(w & 0x3FFF, N - 1) + t * N
            idx2[p, pl.ds(b, 16)] = sidx

            @pl.when(t == 0)
            def _cnt():
              valid = (base + b + lax.iota(jnp.int32, 16)) < cnt_j
              dl16 = (w >> 14) & 511
              plsc.addupdate_scatter(cntv, [dl16],
                                     jnp.ones((16,), jnp.int32), mask=valid)

          pltpu.async_copy(p_hbm.at[idx2.at[p]], rows2.at[p], sems[p])

        def process(kk, p):
          ec = jnp.minimum(CH, cnt_j - kk * CH)

          def egrp(g, _2):
            b16 = pl.multiple_of(g * 16, 16)
            wvec = pk2[p, pl.ds(b16, 16)]
            for u in range(16):
              wv = wvec[u]
              i = b16 + u
              dl = jnp.where(i < ec, jnp.minimum((wv >> 14) & 511, NPW), NPW)
              dlo = dl * H
              ato = ((wv >> 23) & 3) * H
              for cg in range(H // 16):
                co = cg * 16
                m = rows2[p, i, pl.ds(co, 16)] + ct[pl.ds(ato + co, 16)]
                sl = pl.ds(dlo + co, 16)
                accs[sl] = accs[sl] + m
                accq[sl] = accq[sl] + m * m
                accmn[sl] = jnp.minimum(accmn[sl], m)
                accmx[sl] = jnp.maximum(accmx[sl], m)

            return 0

          lax.fori_loop(0, (ec + 15) // 16, egrp, 0)

        @pl.when(nch > 0)
        def _prologue():
          fetch(0, 0)

        def pair(gp, _c):
          g0 = gp * 2
          for p in (0, 1):
            g = g0 + p

            @pl.when(g < nch)
            def _one(g=g, p=p):
              @pl.when(g + 1 < nch)
              def _next():
                fetch(g + 1, 1 - p)

              pltpu.make_async_copy(
                  p_hbm.at[idx2.at[p]], rows2.at[p], sems[p]).wait()
              process(g, p)

          return 0

        lax.fori_loop(0, (nch + 1) // 2, pair, 0)

      pltpu.sync_copy(accs, s_out.at[pl.ds((t * NW + wid) * AW, AW)])
      pltpu.sync_copy(accq, q_out.at[pl.ds((t * NW + wid) * AW, AW)])
      pltpu.sync_copy(accmn, mn_out.at[pl.ds((t * NW + wid) * AW, AW)])
      pltpu.sync_copy(accmx, mx_out.at[pl.ds((t * NW + wid) * AW, AW)])

    pltpu.sync_copy(cntv, deg_out.at[pl.ds(wid * NPW_PAD, NPW_PAD)])

  return k(p_flat, ctab, buckets, counts)


def _node_enc(x_p, w_p, b):
  def body(x_ref, w_ref, b_ref, o_ref):
    o_ref[...] = jnp.dot(x_ref[...], w_ref[...], precision=_PREC) + b_ref[...]

  return pl.pallas_call(
      body,
      out_shape=jax.ShapeDtypeStruct((N, H), jnp.float32),
  )(x_p, w_p, b)


def _ctables(eenc_W, eenc_b, ee_W, ee_b, we4):
  def body(ew_ref, eb_ref, w_ref, b_ref, we_ref, o_ref):
    ea_h = ew_ref[...] + eb_ref[...]
    for l in range(L):
      ea = jnp.dot(ea_h, w_ref[l], precision=_PREC) + b_ref[l][None]
      for t in range(T):
        o_ref[l, t] = jnp.dot(ea, we_ref[l, t], precision=_PREC)

  return pl.pallas_call(
      body,
      out_shape=jax.ShapeDtypeStruct((L, T, 4, H), jnp.float32),
  )(eenc_W, eenc_b.reshape(1, H), ee_W, ee_b, we4)


def _pre(h, wi3, wj3, pre_b):
  def body(h_ref, wi_ref, wj_ref, b_ref, a_ref, p_ref):
    hh = h_ref[...]
    a_ref[0] = jnp.dot(hh, wi_ref[0], precision=_PREC) + b_ref[0]
    pp = jnp.dot(hh, wj_ref[0], precision=_PREC)
    p_ref[0] = jnp.concatenate(
        [pp, jnp.zeros((N, 128 - H), jnp.float32)], axis=-1)

  return pl.pallas_call(
      body,
      grid=(T,),
      in_specs=[
          pl.BlockSpec((N, H), lambda t: (0, 0)),
          pl.BlockSpec((1, H, H), lambda t: (t, 0, 0)),
          pl.BlockSpec((1, H, H), lambda t: (t, 0, 0)),
          pl.BlockSpec((1, 1, H), lambda t: (t, 0, 0)),
      ],
      out_specs=[
          pl.BlockSpec((1, N, H), lambda t: (t, 0, 0)),
          pl.BlockSpec((1, N, 128), lambda t: (t, 0, 0)),
      ],
      out_shape=[
          jax.ShapeDtypeStruct((T, N, H), jnp.float32),
          jax.ShapeDtypeStruct((T, N, 128), jnp.float32),
      ],
  )(h, wi3, wj3, pre_b.reshape(T, 1, H))


def _post(h, a3, s3, q3, mn3, mx3, deg, wx, w1, w2, w3, pb):
  def body(h_ref, a_ref, s_ref, q_ref, mn_ref, mx_ref, d_ref,
           wx_ref, w1_ref, w2_ref, w3_ref, pb_ref, o_ref):
    d = d_ref[...]
    degc = jnp.maximum(d, 1.0)
    a = a_ref[0]
    s = s_ref[0]
    q = q_ref[0]
    s1 = d * a + s
    mean = s1 / degc
    msq = (d * a * a + 2.0 * a * s + q) / degc
    var = jnp.maximum(msq - mean * mean, 0.0)
    std = jnp.sqrt(var + 1e-5)
    has = d > 0.0
    mn = jnp.where(has, a + mn_ref[0], 0.0)
    mx = jnp.where(has, a + mx_ref[0], 0.0)
    dl = jnp.log(degc + 1.0)
    sc1 = dl / AVG_LOG
    sc2 = AVG_LOG / dl
    agg = jnp.concatenate([mean, mn, mx, std], axis=-1)
    out = (jnp.dot(h_ref[...], wx_ref[0], precision=_PREC)
           + jnp.dot(agg, w1_ref[0], precision=_PREC)
           + sc1 * jnp.dot(agg, w2_ref[0], precision=_PREC)
           + sc2 * jnp.dot(agg, w3_ref[0], precision=_PREC)
           + pb_ref[0])
    o_ref[0] = out

  nto = H // T
  nb = 2000
  return pl.pallas_call(
      body,
      grid=(T, N // nb),
      in_specs=[
          pl.BlockSpec((nb, H), lambda t, i: (i, 0)),
          pl.BlockSpec((1, nb, H), lambda t, i: (t, i, 0)),
          pl.BlockSpec((1, nb, H), lambda t, i: (t, i, 0)),
          pl.BlockSpec((1, nb, H), lambda t, i: (t, i, 0)),
          pl.BlockSpec((1, nb, H), lambda t, i: (t, i, 0)),
          pl.BlockSpec((1, nb, H), lambda t, i: (t, i, 0)),
          pl.BlockSpec((nb, 1), lambda t, i: (i, 0)),
          pl.BlockSpec((1, H, nto), lambda t, i: (t, 0, 0)),
          pl.BlockSpec((1, 4 * H, nto), lambda t, i: (t, 0, 0)),
          pl.BlockSpec((1, 4 * H, nto), lambda t, i: (t, 0, 0)),
          pl.BlockSpec((1, 4 * H, nto), lambda t, i: (t, 0, 0)),
          pl.BlockSpec((1, 1, nto), lambda t, i: (t, 0, 0)),
      ],
      out_specs=pl.BlockSpec((1, nb, nto), lambda t, i: (t, i, 0)),
      out_shape=jax.ShapeDtypeStruct((T, N, nto), jnp.float32),
  )(h, a3, s3, q3, mn3, mx3, deg, wx, w1, w2, w3, pb.reshape(T, 1, nto))


def _lin_bn_res(y4, h_in, lw, lb, g, b):
  def body(y_ref, h_ref, w_ref, b_ref, g_ref, bb_ref, o_ref):
    yy = jnp.concatenate([y_ref[t] for t in range(T)], axis=-1)
    y0 = jnp.dot(yy, w_ref[...], precision=_PREC) + b_ref[...]
    m = jnp.mean(y0, axis=0, keepdims=True)
    v = jnp.mean((y0 - m) * (y0 - m), axis=0, keepdims=True)
    hn = (y0 - m) / jnp.sqrt(v + 1e-5) * g_ref[...] + bb_ref[...]
    o_ref[...] = jnp.maximum(hn, 0.0) + h_ref[...]

  return pl.pallas_call(
      body,
      out_shape=jax.ShapeDtypeStruct((N, H), jnp.float32),
  )(y4, h_in, lw, lb.reshape(1, H), g.reshape(1, H), b.reshape(1, H))


def _pool_head(h, batch2, y2, w1, b1, w3, b3):
  def body(h_ref, bt_ref, y_ref, w1_ref, b1_ref, w3_ref, b3_ref, o_ref):
    gids = lax.broadcasted_iota(jnp.int32, (1, G), 1)
    oh = (bt_ref[...] == gids).astype(jnp.float32)
    pooled = lax.dot_general(oh, h_ref[...], (((0,), (0,)), ((), ())),
                             precision=_PREC)
    xc = jnp.maximum(jnp.dot(pooled, w1_ref[...], precision=_PREC)
                     + b1_ref[...], 0.0)
    pred = jnp.dot(xc, w3_ref[...], precision=_PREC) + b3_ref[...]
    dd = jnp.abs(pred - y_ref[...])
    beta = 0.5
    ls = jnp.where(dd < beta, 0.5 * dd * dd / beta, dd - 0.5 * beta)
    o_ref[...] = jnp.mean(ls).reshape(1, 1)

  return pl.pallas_call(
      body,
      out_shape=jax.ShapeDtypeStruct((1, 1), jnp.float32),
  )(h, batch2, y2, w1, b1.reshape(1, H), w3, b3.reshape(1, 1))


def kernel(x, edge_index, edge_attr, batch, y, node_W, node_b, eenc_W,
           eenc_b, pna_ee_W, pna_ee_b, pre_W, pre_b, post_W, post_b, lin_W,
           lin_b, bn_g, bn_b, lin1_W, lin1_b, lin3_W, lin3_b):
  f32 = jnp.float32
  src = edge_index[0].astype(jnp.int32)
  dst = edge_index[1].astype(jnp.int32)
  attr = edge_attr.astype(jnp.int32)

  buckets, counts = _bucketize(src, dst, attr)

  x_p = jnp.pad(x.astype(f32), ((0, 0), (0, 7)))
  w_p = jnp.pad(node_W.astype(f32), ((0, 7), (0, 0)))
  h = _node_enc(x_p, w_p, node_b.astype(f32).reshape(1, H))

  we4 = pre_W[:, :, 2 * H:3 * H, :].astype(f32)
  c_all = _ctables(eenc_W.astype(f32), eenc_b.astype(f32),
                   pna_ee_W.astype(f32), pna_ee_b.astype(f32), we4)

  for l in range(L):
    wi3 = pre_W[l, :, :H, :].astype(f32)
    wj3 = pre_W[l, :, H:2 * H, :].astype(f32)
    a3, p3 = _pre(h, wi3, wj3, pre_b[l].astype(f32))

    s4, q4, mn4, mx4, cnt = _tower_pass(
        p3.reshape(T * N, 128), c_all[l].reshape(-1), buckets, counts)

    def _trim(z):
      z = z.reshape(T, NW, NPW_PAD, H)
      return z[:, :, :NPW].reshape(T, NW * NPW, H)[:, :N]

    s3 = _trim(s4)
    q3 = _trim(q4)
    mn3 = _trim(mn4)
    mx3 = _trim(mx4)
    deg = cnt.reshape(NW, NPW_PAD)[:, :NPW].reshape(-1)[:N]
    deg = deg.astype(f32).reshape(N, 1)

    wx = post_W[l, :, :H, :].astype(f32)
    w1 = post_W[l, :, H:5 * H, :].astype(f32)
    w2 = post_W[l, :, 5 * H:9 * H, :].astype(f32)
    w3 = post_W[l, :, 9 * H:13 * H, :].astype(f32)

    out64 = _post(h, a3, s3, q3, mn3, mx3, deg, wx, w1, w2, w3,
                  post_b[l].astype(f32))
    h = _lin_bn_res(out64, h, lin_W[l].astype(f32), lin_b[l].astype(f32),
                    bn_g[l].astype(f32), bn_b[l].astype(f32))

  loss = _pool_head(h, batch.astype(jnp.int32).reshape(N, 1),
                    y.astype(f32).reshape(G, 1), lin1_W.astype(f32),
                    lin1_b.astype(f32), lin3_W.astype(f32),
                    lin3_b.astype(f32))
  loss = loss.reshape(())
  return (loss, loss)
